# Initial kernel scaffold; baseline (speedup 1.0000x reference)
import jax
import jax.numpy as jnp
from jax.experimental import pallas as pl


def kernel(x, E):
    raise NotImplementedError("write your pallas kernel here")


# speedup vs baseline: 2.3137x; 2.3137x over previous
import functools

import jax
import jax.numpy as jnp
from jax import lax
from jax.experimental import pallas as pl
from jax.experimental.pallas import tpu as pltpu
from jax.experimental.pallas import tpu_sc as plsc

NC = 2
NS = 16
NW = NC * NS

IDX_PER_STREAM = 128
STREAMS_PER_CHUNK = 8
CHUNK_IDX = IDX_PER_STREAM * STREAMS_PER_CHUNK


def _make_kernel(n_idx, emb_dim, n_hash):
    assert n_idx % (NW * CHUNK_IDX) == 0
    idx_per_w = n_idx // NW
    chunks_per_w = idx_per_w // CHUNK_IDX
    out_per_chunk = CHUNK_IDX // n_hash
    out_per_w = idx_per_w // n_hash
    n_out = n_idx // n_hash
    half = emb_dim // 2

    mesh = plsc.VectorSubcoreMesh(
        core_axis_name="c", subcore_axis_name="s", num_cores=NC, num_subcores=NS
    )

    @functools.partial(
        pl.kernel,
        out_type=jax.ShapeDtypeStruct((n_out, emb_dim), jnp.float32),
        mesh=mesh,
        scratch_types=[
            pltpu.VMEM((STREAMS_PER_CHUNK, IDX_PER_STREAM), jnp.int32),
            pltpu.VMEM((CHUNK_IDX, emb_dim), jnp.float32),
            pltpu.VMEM((out_per_chunk, emb_dim), jnp.float32),
            pltpu.SemaphoreType.DMA,
        ],
        compiler_params=pltpu.CompilerParams(use_tc_tiling_on_sc=False),
    )
    def k(x_hbm, e_hbm, y_hbm, idx_v, rows_v, out_v, gsem):
        wid = lax.axis_index("s") * NC + lax.axis_index("c")
        idx_row0 = wid * (idx_per_w // IDX_PER_STREAM)
        out_base = wid * out_per_w

        @pl.loop(0, chunks_per_w)
        def _chunk(c):
            pltpu.sync_copy(
                x_hbm.at[pl.ds(idx_row0 + c * STREAMS_PER_CHUNK, STREAMS_PER_CHUNK)],
                idx_v,
            )
            for j in range(STREAMS_PER_CHUNK):
                pltpu.async_copy(
                    e_hbm.at[idx_v.at[j]],
                    rows_v.at[pl.ds(j * IDX_PER_STREAM, IDX_PER_STREAM)],
                    gsem,
                )
            for j in range(STREAMS_PER_CHUNK):
                pltpu.make_async_copy(
                    e_hbm.at[idx_v.at[j]],
                    rows_v.at[pl.ds(j * IDX_PER_STREAM, IDX_PER_STREAM)],
                    gsem,
                ).wait()

            @pl.loop(0, out_per_chunk, unroll=4)
            def _red(t):
                r = t * n_hash
                lo = rows_v[r, pl.ds(0, half)]
                hi = rows_v[r, pl.ds(half, half)]
                for h in range(1, n_hash):
                    lo = lo + rows_v[r + h, pl.ds(0, half)]
                    hi = hi + rows_v[r + h, pl.ds(half, half)]
                out_v[t, pl.ds(0, half)] = lo
                out_v[t, pl.ds(half, half)] = hi

            pltpu.sync_copy(
                out_v, y_hbm.at[pl.ds(out_base + c * out_per_chunk, out_per_chunk)]
            )

    return k


def kernel(x, E):
    b, l, h = x.shape
    n_tok, emb_dim = E.shape
    n_idx = b * l * h
    x2d = x.reshape(n_idx // IDX_PER_STREAM, IDX_PER_STREAM).astype(jnp.int32)
    y = _make_kernel(n_idx, emb_dim, h)(x2d, E)
    return y.reshape(b, l, emb_dim)


# speedup vs baseline: 2.5540x; 1.1039x over previous
import functools

import jax
import jax.numpy as jnp
from jax import lax
from jax.experimental import pallas as pl
from jax.experimental.pallas import tpu as pltpu
from jax.experimental.pallas import tpu_sc as plsc

NC = 2
NS = 16
NW = NC * NS

IDX_PER_STREAM = 128
STREAMS_PER_CHUNK = 8
CHUNK_IDX = IDX_PER_STREAM * STREAMS_PER_CHUNK


def _make_kernel(n_idx, emb_dim, n_hash):
    assert n_idx % (NW * CHUNK_IDX) == 0
    idx_per_w = n_idx // NW
    chunks = idx_per_w // CHUNK_IDX
    assert chunks >= 4 and chunks % 2 == 0
    out_per_chunk = CHUNK_IDX // n_hash
    out_per_w = idx_per_w // n_hash
    n_out = n_idx // n_hash
    half = emb_dim // 2

    mesh = plsc.VectorSubcoreMesh(
        core_axis_name="c", subcore_axis_name="s", num_cores=NC, num_subcores=NS
    )

    @functools.partial(
        pl.kernel,
        out_type=jax.ShapeDtypeStruct((n_out, emb_dim), jnp.float32),
        mesh=mesh,
        scratch_types=[
            pltpu.VMEM((2, STREAMS_PER_CHUNK, IDX_PER_STREAM), jnp.int32),
            pltpu.VMEM((2, CHUNK_IDX, emb_dim), jnp.float32),
            pltpu.VMEM((2, out_per_chunk, emb_dim), jnp.float32),
            pltpu.SemaphoreType.DMA,
            pltpu.SemaphoreType.DMA,
            pltpu.SemaphoreType.DMA,
            pltpu.SemaphoreType.DMA,
        ],
        compiler_params=pltpu.CompilerParams(use_tc_tiling_on_sc=False),
    )
    def k(x_hbm, e_hbm, y_hbm, idx_v, rows_v, out_v, isem, gsem, osem0, osem1):
        wid = lax.axis_index("s") * NC + lax.axis_index("c")
        idx_row0 = wid * (idx_per_w // IDX_PER_STREAM)
        out_base = wid * out_per_w
        osems = (osem0, osem1)

        def idx_load(c, s):
            return pltpu.async_copy(
                x_hbm.at[pl.ds(idx_row0 + c * STREAMS_PER_CHUNK, STREAMS_PER_CHUNK)],
                idx_v.at[s],
                isem,
            )

        def wait_idx_load(c, s):
            pltpu.make_async_copy(
                x_hbm.at[pl.ds(idx_row0 + c * STREAMS_PER_CHUNK, STREAMS_PER_CHUNK)],
                idx_v.at[s],
                isem,
            ).wait()

        def gathers(s):
            for j in range(STREAMS_PER_CHUNK):
                pltpu.async_copy(
                    e_hbm.at[idx_v.at[s, j]],
                    rows_v.at[s, pl.ds(j * IDX_PER_STREAM, IDX_PER_STREAM)],
                    gsem,
                )

        def wait_gathers(s):
            for j in range(STREAMS_PER_CHUNK):
                pltpu.make_async_copy(
                    e_hbm.at[idx_v.at[s, j]],
                    rows_v.at[s, pl.ds(j * IDX_PER_STREAM, IDX_PER_STREAM)],
                    gsem,
                ).wait()

        def store(c, s):
            return pltpu.async_copy(
                out_v.at[s],
                y_hbm.at[pl.ds(out_base + c * out_per_chunk, out_per_chunk)],
                osems[s],
            )

        def wait_store(c, s):
            pltpu.make_async_copy(
                out_v.at[s],
                y_hbm.at[pl.ds(out_base + c * out_per_chunk, out_per_chunk)],
                osems[s],
            ).wait()

        def reduce(s):
            rv = rows_v.at[s]
            ov = out_v.at[s]

            @pl.loop(0, out_per_chunk, unroll=8)
            def _red(t):
                r = t * n_hash
                lo = rv[r, pl.ds(0, half)]
                hi = rv[r, pl.ds(half, half)]
                for h in range(1, n_hash):
                    lo = lo + rv[r + h, pl.ds(0, half)]
                    hi = hi + rv[r + h, pl.ds(half, half)]
                ov[t, pl.ds(0, half)] = lo
                ov[t, pl.ds(half, half)] = hi

        def step(c, s, issue_next, load_next2, drain_store):
            wait_gathers(s)
            if issue_next:
                wait_idx_load(c + 1, 1 - s)
                gathers(1 - s)
            if load_next2:
                idx_load(c + 2, s)
            if drain_store:
                wait_store(c - 2, s)
            reduce(s)
            store(c, s)

        idx_load(0, 0).wait()
        gathers(0)
        idx_load(1, 1)
        step(0, 0, True, True, False)
        step(1, 1, True, True, False)

        @pl.loop(1, (chunks - 4) // 2 + 1)
        def _main(i):
            step(2 * i, 0, True, True, True)
            step(2 * i + 1, 1, True, True, True)

        step(chunks - 2, 0, True, False, True)
        step(chunks - 1, 1, False, False, True)
        wait_store(chunks - 2, 0)
        wait_store(chunks - 1, 1)

    return k


def kernel(x, E):
    b, l, h = x.shape
    n_tok, emb_dim = E.shape
    n_idx = b * l * h
    x2d = x.reshape(n_idx // IDX_PER_STREAM, IDX_PER_STREAM).astype(jnp.int32)
    y = _make_kernel(n_idx, emb_dim, h)(x2d, E)
    return y.reshape(b, l, emb_dim)


# speedup vs baseline: 2.5569x; 1.0012x over previous
import functools

import jax
import jax.numpy as jnp
from jax import lax
from jax.experimental import pallas as pl
from jax.experimental.pallas import tpu as pltpu
from jax.experimental.pallas import tpu_sc as plsc

NC = 2
NS = 16
NW = NC * NS

IDX_PER_STREAM = 1024
STREAMS_PER_CHUNK = 1
CHUNK_IDX = IDX_PER_STREAM * STREAMS_PER_CHUNK


def _make_kernel(n_idx, emb_dim, n_hash):
    assert n_idx % (NW * CHUNK_IDX) == 0
    idx_per_w = n_idx // NW
    chunks = idx_per_w // CHUNK_IDX
    assert chunks >= 4 and chunks % 2 == 0
    out_per_chunk = CHUNK_IDX // n_hash
    out_per_w = idx_per_w // n_hash
    n_out = n_idx // n_hash
    half = emb_dim // 2

    mesh = plsc.VectorSubcoreMesh(
        core_axis_name="c", subcore_axis_name="s", num_cores=NC, num_subcores=NS
    )

    @functools.partial(
        pl.kernel,
        out_type=jax.ShapeDtypeStruct((n_out, emb_dim), jnp.float32),
        mesh=mesh,
        scratch_types=[
            pltpu.VMEM((2, STREAMS_PER_CHUNK, IDX_PER_STREAM), jnp.int32),
            pltpu.VMEM((2, CHUNK_IDX, emb_dim), jnp.float32),
            pltpu.VMEM((2, out_per_chunk, emb_dim), jnp.float32),
            pltpu.SemaphoreType.DMA,
            pltpu.SemaphoreType.DMA,
            pltpu.SemaphoreType.DMA,
            pltpu.SemaphoreType.DMA,
        ],
        compiler_params=pltpu.CompilerParams(use_tc_tiling_on_sc=False),
    )
    def k(x_hbm, e_hbm, y_hbm, idx_v, rows_v, out_v, isem, gsem, osem0, osem1):
        wid = lax.axis_index("s") * NC + lax.axis_index("c")
        idx_row0 = wid * (idx_per_w // IDX_PER_STREAM)
        out_base = wid * out_per_w
        osems = (osem0, osem1)

        def idx_load(c, s):
            return pltpu.async_copy(
                x_hbm.at[pl.ds(idx_row0 + c * STREAMS_PER_CHUNK, STREAMS_PER_CHUNK)],
                idx_v.at[s],
                isem,
            )

        def wait_idx_load(c, s):
            pltpu.make_async_copy(
                x_hbm.at[pl.ds(idx_row0 + c * STREAMS_PER_CHUNK, STREAMS_PER_CHUNK)],
                idx_v.at[s],
                isem,
            ).wait()

        def gathers(s):
            for j in range(STREAMS_PER_CHUNK):
                pltpu.async_copy(
                    e_hbm.at[idx_v.at[s, j]],
                    rows_v.at[s, pl.ds(j * IDX_PER_STREAM, IDX_PER_STREAM)],
                    gsem,
                )

        def wait_gathers(s):
            for j in range(STREAMS_PER_CHUNK):
                pltpu.make_async_copy(
                    e_hbm.at[idx_v.at[s, j]],
                    rows_v.at[s, pl.ds(j * IDX_PER_STREAM, IDX_PER_STREAM)],
                    gsem,
                ).wait()

        def store(c, s):
            return pltpu.async_copy(
                out_v.at[s],
                y_hbm.at[pl.ds(out_base + c * out_per_chunk, out_per_chunk)],
                osems[s],
            )

        def wait_store(c, s):
            pltpu.make_async_copy(
                out_v.at[s],
                y_hbm.at[pl.ds(out_base + c * out_per_chunk, out_per_chunk)],
                osems[s],
            ).wait()

        def reduce(s):
            rv = rows_v.at[s]
            ov = out_v.at[s]

            @pl.loop(0, out_per_chunk, unroll=8)
            def _red(t):
                r = t * n_hash
                lo = rv[r, pl.ds(0, half)]
                hi = rv[r, pl.ds(half, half)]
                for h in range(1, n_hash):
                    lo = lo + rv[r + h, pl.ds(0, half)]
                    hi = hi + rv[r + h, pl.ds(half, half)]
                ov[t, pl.ds(0, half)] = lo
                ov[t, pl.ds(half, half)] = hi

        def step(c, s, issue_next, load_next2, drain_store):
            wait_gathers(s)
            if issue_next:
                wait_idx_load(c + 1, 1 - s)
                gathers(1 - s)
            if load_next2:
                idx_load(c + 2, s)
            if drain_store:
                wait_store(c - 2, s)
            reduce(s)
            store(c, s)

        idx_load(0, 0).wait()
        gathers(0)
        idx_load(1, 1)
        step(0, 0, True, True, False)
        step(1, 1, True, True, False)

        @pl.loop(1, (chunks - 4) // 2 + 1)
        def _main(i):
            step(2 * i, 0, True, True, True)
            step(2 * i + 1, 1, True, True, True)

        step(chunks - 2, 0, True, False, True)
        step(chunks - 1, 1, False, False, True)
        wait_store(chunks - 2, 0)
        wait_store(chunks - 1, 1)

    return k


def kernel(x, E):
    b, l, h = x.shape
    n_tok, emb_dim = E.shape
    n_idx = b * l * h
    x2d = x.reshape(n_idx // IDX_PER_STREAM, IDX_PER_STREAM).astype(jnp.int32)
    y = _make_kernel(n_idx, emb_dim, h)(x2d, E)
    return y.reshape(b, l, emb_dim)


# speedup vs baseline: 3.3429x; 1.3074x over previous
import functools

import jax
import jax.numpy as jnp
from jax import lax
from jax.experimental import pallas as pl
from jax.experimental.pallas import tpu as pltpu
from jax.experimental.pallas import tpu_sc as plsc

NC = 2
NS = 16
NW = NC * NS

CHUNK_IDX = 1024


def _make_kernel(n_idx, emb_dim, n_hash):
    assert n_idx % (NW * CHUNK_IDX) == 0
    idx_per_w = n_idx // NW
    chunks = idx_per_w // CHUNK_IDX
    assert chunks >= 4 and chunks % 2 == 0
    out_per_chunk = CHUNK_IDX // n_hash
    packed_per_chunk = out_per_chunk * emb_dim // 128
    n_out = n_idx // n_hash
    half = emb_dim // 2

    mesh = plsc.VectorSubcoreMesh(
        core_axis_name="c", subcore_axis_name="s", num_cores=NC, num_subcores=NS
    )

    @functools.partial(
        pl.kernel,
        out_type=jax.ShapeDtypeStruct((n_out * emb_dim // 128, 128), jnp.float32),
        mesh=mesh,
        scratch_types=[
            pltpu.VMEM((2, 8, CHUNK_IDX // 8), jnp.int32),
            pltpu.VMEM((2, CHUNK_IDX, emb_dim), jnp.float32),
            pltpu.VMEM((2, packed_per_chunk, 128), jnp.float32),
            pltpu.SemaphoreType.DMA,
            pltpu.SemaphoreType.DMA,
            pltpu.SemaphoreType.DMA,
            pltpu.SemaphoreType.DMA,
        ],
        compiler_params=pltpu.CompilerParams(use_tc_tiling_on_sc=False),
    )
    def k(x_hbm, e_hbm, y_hbm, idx_v, rows_v, out_v, isem, gsem, osem0, osem1):
        wid = lax.axis_index("s") * NC + lax.axis_index("c")
        idx_row0 = wid * (idx_per_w // 128)
        out_base = wid * (idx_per_w // n_hash * emb_dim // 128)
        osems = (osem0, osem1)

        def idx_load(c, s):
            return pltpu.async_copy(
                x_hbm.at[pl.ds(idx_row0 + c * 8, 8)], idx_v.at[s], isem
            )

        def wait_idx_load(s):
            pltpu.make_async_copy(
                x_hbm.at[pl.ds(idx_row0, 8)], idx_v.at[s], isem
            ).wait()

        def gathers(s):
            for j in range(8):
                pltpu.async_copy(
                    e_hbm.at[idx_v.at[s, j]],
                    rows_v.at[s, pl.ds(j * 128, 128)],
                    gsem,
                )

        def wait_gathers(s):
            for j in range(8):
                pltpu.make_async_copy(
                    e_hbm.at[idx_v.at[s, j]],
                    rows_v.at[s, pl.ds(j * 128, 128)],
                    gsem,
                ).wait()

        def store(c, s):
            return pltpu.async_copy(
                out_v.at[s],
                y_hbm.at[pl.ds(out_base + c * packed_per_chunk, packed_per_chunk)],
                osems[s],
            )

        def wait_store(s):
            pltpu.make_async_copy(
                out_v.at[s],
                y_hbm.at[pl.ds(out_base, packed_per_chunk)],
                osems[s],
            ).wait()

        def reduce(s):
            rv = rows_v.at[s]
            ov = out_v.at[s]

            @pl.loop(0, out_per_chunk, unroll=8)
            def _red(t):
                r = t * n_hash
                lo = rv[r, pl.ds(0, half)]
                hi = rv[r, pl.ds(half, half)]
                for h in range(1, n_hash):
                    lo = lo + rv[r + h, pl.ds(0, half)]
                    hi = hi + rv[r + h, pl.ds(half, half)]
                pr = t // 4
                pc = (t % 4) * emb_dim
                ov[pr, pl.ds(pc, half)] = lo
                ov[pr, pl.ds(pc + half, half)] = hi

        def step(c, s, issue_next, load_next2, drain_store):
            wait_gathers(s)
            if issue_next:
                wait_idx_load(1 - s)
                gathers(1 - s)
            if load_next2:
                idx_load(c + 2, s)
            if drain_store:
                wait_store(s)
            reduce(s)
            store(c, s)

        idx_load(0, 0)
        wait_idx_load(0)
        gathers(0)
        idx_load(1, 1)
        step(0, 0, True, True, False)
        step(1, 1, True, True, False)

        @pl.loop(1, (chunks - 4) // 2 + 1)
        def _main(i):
            step(2 * i, 0, True, True, True)
            step(2 * i + 1, 1, True, True, True)

        step(chunks - 2, 0, True, False, True)
        step(chunks - 1, 1, False, False, True)
        wait_store(0)
        wait_store(1)

    return k


def kernel(x, E):
    b, l, h = x.shape
    n_tok, emb_dim = E.shape
    n_idx = b * l * h
    x2d = x.reshape(n_idx // 128, 128).astype(jnp.int32)
    y = _make_kernel(n_idx, emb_dim, h)(x2d, E)
    return y.reshape(b, l, emb_dim)


# speedup vs baseline: 6.2172x; 1.8598x over previous
import functools

import jax
import jax.numpy as jnp
from jax import lax
from jax.experimental import pallas as pl
from jax.experimental.pallas import tpu as pltpu
from jax.experimental.pallas import tpu_sc as plsc

NC = 2
NS = 16
NW = NC * NS

CHUNK_IDX = 1024


def _make_kernel(n_idx, emb_dim, n_hash):
    assert n_idx % (NW * CHUNK_IDX) == 0
    idx_per_w = n_idx // NW
    chunks = idx_per_w // CHUNK_IDX
    assert chunks >= 4 and chunks % 2 == 0
    out_per_chunk = CHUNK_IDX // n_hash
    packed_per_chunk = out_per_chunk * emb_dim // 128
    n_out = n_idx // n_hash
    half = emb_dim // 2

    mesh = plsc.VectorSubcoreMesh(
        core_axis_name="c", subcore_axis_name="s", num_cores=NC, num_subcores=NS
    )

    @functools.partial(
        pl.kernel,
        out_type=jax.ShapeDtypeStruct((n_out * emb_dim // 128, 128), jnp.float32),
        mesh=mesh,
        scratch_types=[
            pltpu.VMEM((2, 8, CHUNK_IDX // 8), jnp.int32),
            pltpu.VMEM((2, CHUNK_IDX, emb_dim), jnp.float32),
            pltpu.VMEM((2, packed_per_chunk, 128), jnp.float32),
            pltpu.SemaphoreType.DMA,
            pltpu.SemaphoreType.DMA,
            pltpu.SemaphoreType.DMA,
            pltpu.SemaphoreType.DMA,
        ],
        compiler_params=pltpu.CompilerParams(use_tc_tiling_on_sc=False),
    )
    def k(x_hbm, e_hbm, y_hbm, idx_v, rows_v, out_v, isem, gsem, osem0, osem1):
        wid = lax.axis_index("s") * NC + lax.axis_index("c")
        idx_row0 = wid * (idx_per_w // 128)
        blk0 = wid * (idx_per_w // 512)
        osems = (osem0, osem1)

        def idx_load(c, s):
            return pltpu.async_copy(
                x_hbm.at[pl.ds(idx_row0 + c * 8, 8)], idx_v.at[s], isem
            )

        def wait_idx_load(s):
            pltpu.make_async_copy(
                x_hbm.at[pl.ds(idx_row0, 8)], idx_v.at[s], isem
            ).wait()

        def gathers(s):
            for j in range(8):
                pltpu.async_copy(
                    e_hbm.at[idx_v.at[s, j]],
                    rows_v.at[s, pl.ds(j * 128, 128)],
                    gsem,
                )

        def wait_gathers(s):
            for j in range(8):
                pltpu.make_async_copy(
                    e_hbm.at[idx_v.at[s, j]],
                    rows_v.at[s, pl.ds(j * 128, 128)],
                    gsem,
                ).wait()

        def store(c, s):
            for half in range(2):
                bk = blk0 + 2 * c + half
                base = (bk // 128) * 4096 + (bk % 128) * 32
                pltpu.async_copy(
                    out_v.at[s, pl.ds(half * 32, 32)],
                    y_hbm.at[pl.ds(base, 32)],
                    osems[s],
                )

        def wait_store(s):
            for half in range(2):
                pltpu.make_async_copy(
                    out_v.at[s, pl.ds(half * 32, 32)],
                    y_hbm.at[pl.ds(0, 32)],
                    osems[s],
                ).wait()

        def reduce(s):
            rv = rows_v.at[s]
            ov = out_v.at[s]

            @pl.loop(0, out_per_chunk, unroll=8)
            def _red(t):
                k = t // 128
                b = t % 128
                r = k * 512 + b
                lo = rv[r, pl.ds(0, half)]
                hi = rv[r, pl.ds(half, half)]
                for h in range(1, n_hash):
                    lo = lo + rv[r + h * 128, pl.ds(0, half)]
                    hi = hi + rv[r + h * 128, pl.ds(half, half)]
                pr = k * 32 + b // 4
                pc = (b % 4) * emb_dim
                ov[pr, pl.ds(pc, half)] = lo
                ov[pr, pl.ds(pc + half, half)] = hi

        def step(c, s, issue_next, load_next2, drain_store):
            wait_gathers(s)
            if issue_next:
                wait_idx_load(1 - s)
                gathers(1 - s)
            if load_next2:
                idx_load(c + 2, s)
            if drain_store:
                wait_store(s)
            reduce(s)
            store(c, s)

        idx_load(0, 0)
        wait_idx_load(0)
        gathers(0)
        idx_load(1, 1)
        step(0, 0, True, True, False)
        step(1, 1, True, True, False)

        @pl.loop(1, (chunks - 4) // 2 + 1)
        def _main(i):
            step(2 * i, 0, True, True, True)
            step(2 * i + 1, 1, True, True, True)

        step(chunks - 2, 0, True, False, True)
        step(chunks - 1, 1, False, False, True)
        wait_store(0)
        wait_store(1)

    return k


def kernel(x, E):
    b, l, h = x.shape
    n_tok, emb_dim = E.shape
    n_idx = b * l * h
    x2d = (
        x.reshape(b // 128, 128, l, h)
        .transpose(2, 0, 3, 1)
        .reshape(n_idx // 128, 128)
        .astype(jnp.int32)
    )
    y = _make_kernel(n_idx, emb_dim, h)(x2d, E)
    return y.reshape(l, b, emb_dim).transpose(1, 0, 2)
